# baseline (speedup 1.0000x reference)
import functools
import math
import numpy as np

import jax
import jax.numpy as jnp
from jax import lax
from jax.experimental import pallas as pl
from jax.experimental.pallas import tpu as pltpu

_LX = 192
_C = 8
_NMARK = 4
_DM = 256
_DB = 128
_DFFN = 512
_H = 4
_DK = 64
_DV = 64
_NL = 3
_WS = (4, 4, 4)
_INNER = 5
_PREDN = 96 * _C
_S = 256
_BT = 16
_VH = 80
_VS = 4 * 80
_LOG2E = math.log2(math.e)


def _static_geometry():
    sizes = [_LX]
    for w in _WS:
        sizes.append(sizes[-1] // w)
    cum = [0]
    for s in sizes:
        cum.append(cum[-1] + s)
    ltot = cum[-1]

    allow = np.zeros((ltot, ltot), dtype=bool)
    iw = _INNER // 2
    for li, sz in enumerate(sizes):
        st = cum[li]
        for i in range(st, st + sz):
            lo = max(i - iw, st)
            hi = min(i + iw + 1, st + sz)
            allow[i, lo:hi] = True
    for li in range(1, len(sizes)):
        st = cum[li]
        for i in range(st, st + sizes[li]):
            lo = (st - sizes[li - 1]) + (i - st) * _WS[li - 1]
            if i == st + sizes[li] - 1:
                hi = st
            else:
                hi = (st - sizes[li - 1]) + (i - st + 1) * _WS[li - 1]
            allow[i, lo:hi] = True
            allow[lo:hi, i] = True

    bias = np.full((_S, _S), -1e9, dtype=np.float32)
    bias[:ltot, :ltot] = np.where(allow, 0.0, -1e9)

    former = sizes[0] - 1
    g_offs = [former]
    for j in range(1, len(sizes)):
        start = cum[j]
        inner = former - (start - sizes[j - 1])
        former = start + min(inner // _WS[j - 1], sizes[j] - 1)
        g_offs.append(former)

    tap_off = tuple(int(v) for v in np.cumsum((0,) + _WS)[:-1])
    return bias, tuple(sizes), tuple(cum), tuple(g_offs), tap_off


_MASK_NP, _SIZES, _CUM, _GOFFS, _TAPOFF = _static_geometry()
_NLVL = len(_SIZES)


def _ln(x, g, b, eps):
    mu = jnp.mean(x, axis=-1, keepdims=True)
    ms = jnp.mean(x * x, axis=-1, keepdims=True)
    var = ms - mu * mu
    return (x - mu) * lax.rsqrt(var + eps) * g + b


def _gelu_tanh(x):
    c = 0.7978845608028654
    return 0.5 * x * (1.0 + jnp.tanh(c * (x + 0.044715 * x * x * x)))


def _fused_kernel(
        xcat_ref, emb_bias_ref, mask_ref,
        wcat_ref, down_w_ref, down_b_ref, conv_w_ref, conv_scale_ref,
        conv_shift_ref, up_w_ref, up_b_ref, cscm_g_ref, cscm_b_ref,
        wqkv_ref, fc_w_ref, fc_b_ref, ln1_g_ref, ln1_b_ref,
        w1_ref, b1_ref, w2_ref, b2_ref, ln2_g_ref, ln2_b_ref,
        pred_w_ref,
        out_ref,
        seq_scr, seqb_scr, x_scr, qkt_scr, vone_scr, gth_scr,
        *, bt):
    f32 = jnp.float32
    bf16 = jnp.bfloat16

    emb = (jnp.dot(xcat_ref[...], wcat_ref[...], preferred_element_type=f32)
           + emb_bias_ref[...])
    for s in range(bt):
        seq_scr[s * _S:s * _S + _LX, :] = emb[s * _LX:(s + 1) * _LX, :]
        seq_scr[s * _S + _S - 1:s * _S + _S, :] = jnp.zeros((1, _DM), f32)

    x_scr[...] = (jnp.dot(emb.astype(bf16), down_w_ref[...],
                          preferred_element_type=f32) + down_b_ref[...])
    rows_in = bt * _LX
    for lvl, w in enumerate(_WS):
        rows_out = rows_in // w
        acc = None
        for j in range(w):
            xj = x_scr[pl.ds(j, rows_out, stride=w), :].astype(bf16)
            yj = jnp.dot(xj, conv_w_ref[_TAPOFF[lvl] + j],
                         preferred_element_type=f32)
            acc = yj if acc is None else acc + yj
        y = acc * conv_scale_ref[lvl] + conv_shift_ref[lvl]
        y = jnp.where(y > 0.0, y, jnp.exp(y) - 1.0)
        if lvl + 1 < len(_WS):
            x_scr[0:rows_out, :] = y
        up = (jnp.dot(y.astype(bf16), up_w_ref[...],
                      preferred_element_type=f32) + up_b_ref[...])
        seg = rows_out // bt
        for s in range(bt):
            so = s * _S + _CUM[lvl + 1]
            seq_scr[so:so + seg, :] = up[s * seg:(s + 1) * seg, :]
        rows_in = rows_out

    for s in range(bt):
        v = _ln(seq_scr[s * _S:(s + 1) * _S, :], cscm_g_ref[...],
                cscm_b_ref[...], 1e-5)
        seq_scr[s * _S:(s + 1) * _S, :] = v
        seqb_scr[s * _S:(s + 1) * _S, :] = v.astype(bf16)

    for s in range(bt):
        for h in range(_H):
            vone_scr[s * _VS + h * _VH + _DV:s * _VS + h * _VH + _DV + 1,
                     :] = jnp.ones((1, _S), bf16)

    mask = mask_ref[...]
    for l in range(_NL):
        for s in range(bt):
            r0 = s * _S
            qkvt = lax.dot_general(wqkv_ref[l], seqb_scr[r0:r0 + _S, :],
                                   (((0,), (1,)), ((), ())),
                                   preferred_element_type=f32)
            qkt_scr[s * 2 * _DM:s * 2 * _DM + 2 * _DM, :] = qkvt[
                :2 * _H * _DK, :].astype(bf16)
            for h in range(_H):
                vone_scr[s * _VS + h * _VH:s * _VS + h * _VH + _DV, :] = (
                    qkvt[2 * _H * _DK + h * _DV:
                         2 * _H * _DK + (h + 1) * _DV, :].astype(bf16))
        for s in range(bt):
            q0 = s * 2 * _DM
            a = None
            for h in range(_H):
                qt = qkt_scr[q0 + h * _DK:q0 + (h + 1) * _DK, :]
                kt = qkt_scr[q0 + _H * _DK + h * _DK:
                             q0 + _H * _DK + (h + 1) * _DK, :]
                sct = lax.dot_general(kt, qt, (((0,), (0,)), ((), ())),
                                      preferred_element_type=f32)
                pt = jnp.exp2(jnp.clip(sct + mask, -60.0, 60.0))
                pvt = jnp.dot(
                    vone_scr[s * _VS + h * _VH:s * _VS + (h + 1) * _VH, :],
                    pt.astype(bf16),
                    preferred_element_type=f32)
                aht = (pvt[:_DV, :] * pl.reciprocal(pvt[_DV:_DV + 1, :],
                                                    approx=True)).astype(bf16)
                zh = lax.dot_general(aht,
                                     fc_w_ref[l][h * _DV:(h + 1) * _DV, :],
                                     (((0,), (0,)), ((), ())),
                                     preferred_element_type=f32)
                a = zh if a is None else a + zh
            a = a + fc_b_ref[l]
            res = seq_scr[s * _S:s * _S + _S, :]
            x1 = _ln(a + res, ln1_g_ref[l], ln1_b_ref[l], 1e-6)
            h1 = (jnp.dot(x1.astype(bf16), w1_ref[l],
                          preferred_element_type=f32) + b1_ref[l])
            h1 = _gelu_tanh(h1)
            h2 = (jnp.dot(h1.astype(bf16), w2_ref[l],
                          preferred_element_type=f32) + b2_ref[l])
            o = _ln(h2 + x1, ln2_g_ref[l], ln2_b_ref[l], 1e-6)
            seq_scr[s * _S:s * _S + _S, :] = o
            seqb_scr[s * _S:s * _S + _S, :] = o.astype(bf16)

    for s in range(bt):
        for j in range(_NLVL):
            r = s * _S + _GOFFS[j]
            gth_scr[s:s + 1, j * _DM:(j + 1) * _DM] = seq_scr[r:r + 1, :]
    out_ref[...] = jnp.dot(gth_scr[...], pred_w_ref[...],
                           preferred_element_type=f32)


def kernel(tok_w, temp_w, emb_bias, down_w, down_b, conv_w, conv_scale,
           conv_shift, up_w, up_b, cscm_g, cscm_b, wqkv, fc_w, fc_b,
           ln1_g, ln1_b, w1, b1, w2, b2, ln2_g, ln2_b, pred_w,
           x_enc, x_mark_enc):
    f32 = jnp.float32
    bf16 = jnp.bfloat16
    bt = _BT

    B = x_enc.shape[0]
    x = x_enc.astype(f32)
    xm = x_mark_enc.astype(f32)
    pad = (-B) % bt
    if pad:
        x = jnp.concatenate([x, jnp.zeros((pad, _LX, _C), f32)], axis=0)
        xm = jnp.concatenate([xm, jnp.zeros((pad, _LX, _NMARK), f32)], axis=0)
    Bp = B + pad
    n_blocks = Bp // bt

    xb = x.astype(bf16)
    xp = jnp.stack([jnp.roll(xb, 1, axis=1), xb, jnp.roll(xb, -1, axis=1)],
                   axis=-1).reshape(Bp * _LX, _C * 3)
    xcat = jnp.concatenate([xp, xm.astype(bf16).reshape(Bp * _LX, _NMARK)],
                           axis=1)

    wcat = jnp.concatenate([tok_w, temp_w], axis=0).astype(bf16)
    emb_bias_t = jnp.tile(emb_bias[:_LX], (bt, 1))
    mask = jnp.asarray(_MASK_NP * np.float32(_LOG2E))

    wqkv_s = jnp.concatenate(
        [wqkv[:, :, :_H * _DK] * np.float32(_LOG2E), wqkv[:, :, _H * _DK:]],
        axis=2)

    w_inputs = (
        wcat, down_w.astype(bf16), down_b.astype(f32),
        conv_w.astype(bf16), conv_scale, conv_shift,
        up_w.astype(bf16), up_b, cscm_g, cscm_b,
        wqkv_s.astype(bf16), fc_w.astype(bf16), fc_b, ln1_g, ln1_b,
        w1.astype(bf16), b1, w2.astype(bf16), b2, ln2_g, ln2_b,
        pred_w,
    )

    def _bcast(a):
        nd = a.ndim
        return pl.BlockSpec(tuple(a.shape), lambda g, _n=nd: (0,) * _n)

    in_specs = (
        [pl.BlockSpec((bt * _LX, _C * 3 + _NMARK), lambda g: (g, 0)),
         _bcast(emb_bias_t), _bcast(mask)]
        + [_bcast(a) for a in w_inputs])

    rows_seq = bt * _S
    kernel_fn = functools.partial(_fused_kernel, bt=bt)

    mm = (2 * bt * _LX * 28 * _DM + 2 * bt * _LX * _DM * _DB
          + _NL * bt * (2 * _S * _DM * _H * (2 * _DK + _DV)
                        + 2 * _H * _S * _S * (_DK + 2 * _DV)
                        + 2 * _S * _H * _DV * _DM
                        + 4 * _S * _DM * _DFFN)
          + 2 * bt * _NLVL * _DM * _PREDN)
    trans = _NL * bt * (_H * _S * _S + _S * _DFFN)
    cost = pl.CostEstimate(
        flops=int(n_blocks * mm),
        transcendentals=int(n_blocks * trans),
        bytes_accessed=int(n_blocks * bt * _LX * (_C * 3 + _NMARK) * 2
                           + 20_000_000))

    out = pl.pallas_call(
        kernel_fn,
        out_shape=jax.ShapeDtypeStruct((n_blocks, bt, _PREDN), f32),
        grid=(n_blocks,),
        in_specs=in_specs,
        out_specs=pl.BlockSpec((None, bt, _PREDN), lambda g: (g, 0, 0)),
        scratch_shapes=[
            pltpu.VMEM((rows_seq, _DM), f32),
            pltpu.VMEM((rows_seq, _DM), bf16),
            pltpu.VMEM((bt * _LX, _DB), f32),
            pltpu.VMEM((bt * 2 * _DM, _S), bf16),
            pltpu.VMEM((bt * _VS, _S), bf16),
            pltpu.VMEM((bt, _NLVL * _DM), f32),
        ],
        compiler_params=pltpu.CompilerParams(
            dimension_semantics=("parallel",)),
        cost_estimate=cost,
    )(xcat, emb_bias_t, mask, *w_inputs)

    out = out.reshape(Bp, _PREDN)[:B]
    return out.reshape(B, 96, _C)


# speedup vs baseline: 1.1231x; 1.1231x over previous
import functools
import math
import numpy as np

import jax
import jax.numpy as jnp
from jax import lax
from jax.experimental import pallas as pl
from jax.experimental.pallas import tpu as pltpu

_LX = 192
_C = 8
_NMARK = 4
_DM = 256
_DB = 128
_DFFN = 512
_H = 4
_DK = 64
_DV = 64
_NL = 3
_WS = (4, 4, 4)
_INNER = 5
_PREDN = 96 * _C
_S = 256
_BT = 8
_VH = 80
_VS = 4 * 80
_LOG2E = math.log2(math.e)


def _static_geometry():
    sizes = [_LX]
    for w in _WS:
        sizes.append(sizes[-1] // w)
    cum = [0]
    for s in sizes:
        cum.append(cum[-1] + s)
    ltot = cum[-1]

    allow = np.zeros((ltot, ltot), dtype=bool)
    iw = _INNER // 2
    for li, sz in enumerate(sizes):
        st = cum[li]
        for i in range(st, st + sz):
            lo = max(i - iw, st)
            hi = min(i + iw + 1, st + sz)
            allow[i, lo:hi] = True
    for li in range(1, len(sizes)):
        st = cum[li]
        for i in range(st, st + sizes[li]):
            lo = (st - sizes[li - 1]) + (i - st) * _WS[li - 1]
            if i == st + sizes[li] - 1:
                hi = st
            else:
                hi = (st - sizes[li - 1]) + (i - st + 1) * _WS[li - 1]
            allow[i, lo:hi] = True
            allow[lo:hi, i] = True

    bias = np.full((_S, _S), -1e9, dtype=np.float32)
    bias[:ltot, :ltot] = np.where(allow, 0.0, -1e9)

    former = sizes[0] - 1
    g_offs = [former]
    for j in range(1, len(sizes)):
        start = cum[j]
        inner = former - (start - sizes[j - 1])
        former = start + min(inner // _WS[j - 1], sizes[j] - 1)
        g_offs.append(former)

    tap_off = tuple(int(v) for v in np.cumsum((0,) + _WS)[:-1])
    return bias, tuple(sizes), tuple(cum), tuple(g_offs), tap_off


_MASK_NP, _SIZES, _CUM, _GOFFS, _TAPOFF = _static_geometry()
_NLVL = len(_SIZES)


def _ln(x, g, b, eps):
    mu = jnp.mean(x, axis=-1, keepdims=True)
    ms = jnp.mean(x * x, axis=-1, keepdims=True)
    var = ms - mu * mu
    return (x - mu) * lax.rsqrt(var + eps) * g + b


def _gelu_tanh(x):
    c1 = 0.7978845608028654
    c2 = c1 * 0.044715
    x2 = x * x
    u = x * (c2 * x2 + c1)
    hx = 0.5 * x
    return hx + hx * jnp.tanh(u)


def _fused_kernel(
        xcat_ref, emb_bias_ref, mask_ref,
        wcat_ref, down_w_ref, down_b_ref, conv_w_ref, conv_scale_ref,
        conv_shift_ref, up_w_ref, up_b_ref, cscm_g_ref, cscm_b_ref,
        wqkv_ref, fc_w_ref, fc_b_ref, ln1_g_ref, ln1_b_ref,
        w1_ref, b1_ref, w2_ref, b2_ref, ln2_g_ref, ln2_b_ref,
        pred_w_ref,
        out_ref,
        seq_scr, seqb_scr, x_scr, qkt_scr, vone_scr, gth_scr,
        *, bt):
    f32 = jnp.float32
    bf16 = jnp.bfloat16

    emb = (jnp.dot(xcat_ref[...], wcat_ref[...], preferred_element_type=f32)
           + emb_bias_ref[...])
    for s in range(bt):
        seq_scr[s * _S:s * _S + _LX, :] = emb[s * _LX:(s + 1) * _LX, :]
        seq_scr[s * _S + _S - 1:s * _S + _S, :] = jnp.zeros((1, _DM), f32)

    x_scr[...] = (jnp.dot(emb.astype(bf16), down_w_ref[...],
                          preferred_element_type=f32) + down_b_ref[...])
    rows_in = bt * _LX
    for lvl, w in enumerate(_WS):
        rows_out = rows_in // w
        acc = None
        for j in range(w):
            xj = x_scr[pl.ds(j, rows_out, stride=w), :].astype(bf16)
            yj = jnp.dot(xj, conv_w_ref[_TAPOFF[lvl] + j],
                         preferred_element_type=f32)
            acc = yj if acc is None else acc + yj
        y = acc * conv_scale_ref[lvl] + conv_shift_ref[lvl]
        y = jnp.where(y > 0.0, y, jnp.exp(y) - 1.0)
        if lvl + 1 < len(_WS):
            x_scr[0:rows_out, :] = y
        up = (jnp.dot(y.astype(bf16), up_w_ref[...],
                      preferred_element_type=f32) + up_b_ref[...])
        seg = rows_out // bt
        for s in range(bt):
            so = s * _S + _CUM[lvl + 1]
            seq_scr[so:so + seg, :] = up[s * seg:(s + 1) * seg, :]
        rows_in = rows_out

    for s in range(bt):
        v = _ln(seq_scr[s * _S:(s + 1) * _S, :], cscm_g_ref[...],
                cscm_b_ref[...], 1e-5)
        seq_scr[s * _S:(s + 1) * _S, :] = v
        seqb_scr[s * _S:(s + 1) * _S, :] = v.astype(bf16)

    for s in range(bt):
        for h in range(_H):
            vone_scr[s * _VS + h * _VH + _DV:s * _VS + h * _VH + _DV + 1,
                     :] = jnp.ones((1, _S), bf16)

    mask = mask_ref[...]
    for l in range(_NL):
        for s in range(bt):
            r0 = s * _S
            qkvt = lax.dot_general(wqkv_ref[l], seqb_scr[r0:r0 + _S, :],
                                   (((0,), (1,)), ((), ())),
                                   preferred_element_type=f32)
            qkt_scr[s * 2 * _DM:s * 2 * _DM + 2 * _DM, :] = qkvt[
                :2 * _H * _DK, :].astype(bf16)
            for h in range(_H):
                vone_scr[s * _VS + h * _VH:s * _VS + h * _VH + _DV, :] = (
                    qkvt[2 * _H * _DK + h * _DV:
                         2 * _H * _DK + (h + 1) * _DV, :].astype(bf16))
        for s in range(bt):
            q0 = s * 2 * _DM
            a = None
            for h in range(_H):
                qt = qkt_scr[q0 + h * _DK:q0 + (h + 1) * _DK, :]
                kt = qkt_scr[q0 + _H * _DK + h * _DK:
                             q0 + _H * _DK + (h + 1) * _DK, :]
                sct = lax.dot_general(kt, qt, (((0,), (0,)), ((), ())),
                                      preferred_element_type=f32)
                pt = jnp.exp2(jnp.clip(sct + mask, -60.0, 60.0))
                pvt = jnp.dot(
                    vone_scr[s * _VS + h * _VH:s * _VS + (h + 1) * _VH, :],
                    pt.astype(bf16),
                    preferred_element_type=f32)
                aht = (pvt[:_DV, :] * pl.reciprocal(pvt[_DV:_DV + 1, :],
                                                    approx=True)).astype(bf16)
                zh = lax.dot_general(aht,
                                     fc_w_ref[l][h * _DV:(h + 1) * _DV, :],
                                     (((0,), (0,)), ((), ())),
                                     preferred_element_type=f32)
                a = zh if a is None else a + zh
            a = a + fc_b_ref[l]
            res = seq_scr[s * _S:s * _S + _S, :]
            x1 = _ln(a + res, ln1_g_ref[l], ln1_b_ref[l], 1e-6)
            h1 = (jnp.dot(x1.astype(bf16), w1_ref[l],
                          preferred_element_type=f32) + b1_ref[l])
            h1 = _gelu_tanh(h1)
            h2 = (jnp.dot(h1.astype(bf16), w2_ref[l],
                          preferred_element_type=f32) + b2_ref[l])
            o = _ln(h2 + x1, ln2_g_ref[l], ln2_b_ref[l], 1e-6)
            seq_scr[s * _S:s * _S + _S, :] = o
            seqb_scr[s * _S:s * _S + _S, :] = o.astype(bf16)

    for s in range(bt):
        for j in range(_NLVL):
            r = s * _S + _GOFFS[j]
            gth_scr[s:s + 1, j * _DM:(j + 1) * _DM] = seq_scr[r:r + 1, :]
    out_ref[...] = jnp.dot(gth_scr[...], pred_w_ref[...],
                           preferred_element_type=f32)


def kernel(tok_w, temp_w, emb_bias, down_w, down_b, conv_w, conv_scale,
           conv_shift, up_w, up_b, cscm_g, cscm_b, wqkv, fc_w, fc_b,
           ln1_g, ln1_b, w1, b1, w2, b2, ln2_g, ln2_b, pred_w,
           x_enc, x_mark_enc):
    f32 = jnp.float32
    bf16 = jnp.bfloat16
    bt = _BT

    B = x_enc.shape[0]
    x = x_enc.astype(f32)
    xm = x_mark_enc.astype(f32)
    pad = (-B) % bt
    if pad:
        x = jnp.concatenate([x, jnp.zeros((pad, _LX, _C), f32)], axis=0)
        xm = jnp.concatenate([xm, jnp.zeros((pad, _LX, _NMARK), f32)], axis=0)
    Bp = B + pad
    n_blocks = Bp // bt

    xb = x.astype(bf16)
    xp = jnp.stack([jnp.roll(xb, 1, axis=1), xb, jnp.roll(xb, -1, axis=1)],
                   axis=-1).reshape(Bp * _LX, _C * 3)
    xcat = jnp.concatenate([xp, xm.astype(bf16).reshape(Bp * _LX, _NMARK)],
                           axis=1)

    wcat = jnp.concatenate([tok_w, temp_w], axis=0).astype(bf16)
    emb_bias_t = jnp.tile(emb_bias[:_LX], (bt, 1))
    mask = jnp.asarray(_MASK_NP * np.float32(_LOG2E))

    wqkv_s = jnp.concatenate(
        [wqkv[:, :, :_H * _DK] * np.float32(_LOG2E), wqkv[:, :, _H * _DK:]],
        axis=2)

    w_inputs = (
        wcat, down_w.astype(bf16), down_b.astype(f32),
        conv_w.astype(bf16), conv_scale, conv_shift,
        up_w.astype(bf16), up_b, cscm_g, cscm_b,
        wqkv_s.astype(bf16), fc_w.astype(bf16), fc_b, ln1_g, ln1_b,
        w1.astype(bf16), b1, w2.astype(bf16), b2, ln2_g, ln2_b,
        pred_w,
    )

    def _bcast(a):
        nd = a.ndim
        return pl.BlockSpec(tuple(a.shape), lambda g, _n=nd: (0,) * _n)

    in_specs = (
        [pl.BlockSpec((bt * _LX, _C * 3 + _NMARK), lambda g: (g, 0)),
         _bcast(emb_bias_t), _bcast(mask)]
        + [_bcast(a) for a in w_inputs])

    rows_seq = bt * _S
    kernel_fn = functools.partial(_fused_kernel, bt=bt)

    mm = (2 * bt * _LX * 28 * _DM + 2 * bt * _LX * _DM * _DB
          + _NL * bt * (2 * _S * _DM * _H * (2 * _DK + _DV)
                        + 2 * _H * _S * _S * (_DK + 2 * _DV)
                        + 2 * _S * _H * _DV * _DM
                        + 4 * _S * _DM * _DFFN)
          + 2 * bt * _NLVL * _DM * _PREDN)
    trans = _NL * bt * (_H * _S * _S + _S * _DFFN)
    cost = pl.CostEstimate(
        flops=int(n_blocks * mm),
        transcendentals=int(n_blocks * trans),
        bytes_accessed=int(n_blocks * bt * _LX * (_C * 3 + _NMARK) * 2
                           + 20_000_000))

    out = pl.pallas_call(
        kernel_fn,
        out_shape=jax.ShapeDtypeStruct((n_blocks, bt, _PREDN), f32),
        grid=(n_blocks,),
        in_specs=in_specs,
        out_specs=pl.BlockSpec((None, bt, _PREDN), lambda g: (g, 0, 0)),
        scratch_shapes=[
            pltpu.VMEM((rows_seq, _DM), f32),
            pltpu.VMEM((rows_seq, _DM), bf16),
            pltpu.VMEM((bt * _LX, _DB), f32),
            pltpu.VMEM((bt * 2 * _DM, _S), bf16),
            pltpu.VMEM((bt * _VS, _S), bf16),
            pltpu.VMEM((bt, _NLVL * _DM), f32),
        ],
        compiler_params=pltpu.CompilerParams(
            dimension_semantics=("parallel",)),
        cost_estimate=cost,
    )(xcat, emb_bias_t, mask, *w_inputs)

    out = out.reshape(Bp, _PREDN)[:B]
    return out.reshape(B, 96, _C)
